# Initial kernel scaffold; baseline (speedup 1.0000x reference)
import jax
import jax.numpy as jnp
from jax.experimental import pallas as pl


def kernel(graph, q, k, v, edge_feat, H, W_e2i, W_n2h_q, W_n2h_k, W_n2h_v, W_n2h_o, W_h2n_q, W_h2n_k, W_h2n_v, W_h2n_o, W_o, W_ffn1, b_ffn1, W_ffn2, b_ffn2, W_res, bn1_g, bn1_b, bn2_g, bn2_b):
    raise NotImplementedError("write your pallas kernel here")


# speedup vs baseline: 2412.7222x; 2412.7222x over previous
import jax
import jax.numpy as jnp
from jax.experimental import pallas as pl

_HEADS = 4
_DH = 16
_SCALE = 1.0 / (_DH ** 0.5)


def _hgt_kernel(q_ref, k_ref, H_ref, Ht_ref, ef_ref,
                we2i_ref, wq1_ref, wk1_ref, wv1_ref, wo1_ref,
                wq2_ref, wk2_ref, wv2_ref, wo2_ref,
                wo_ref, wffn1_ref, bffn1_ref, wffn2_ref, bffn2_ref,
                wres_ref, g1_ref, b1_ref, g2_ref, b2_ref,
                out_ref):
    q = q_ref[...]
    k = k_ref[...]
    H = H_ref[...]
    Ht = Ht_ref[...]
    oc = wq1_ref.shape[1]
    neg_inf = float("-inf")

    hsel = jax.lax.broadcasted_iota(jnp.int32, (1, oc), 1) // _DH

    ef = ef_ref[...] @ we2i_ref[...]
    khw = ef @ wk1_ref[...]

    Qn = k @ wq1_ref[...]
    Vn = k @ wv1_ref[...]
    he_upd = jnp.zeros_like(khw)
    for h in range(_HEADS):
        mh = (hsel == h).astype(jnp.float32)
        s = jax.lax.dot_general((khw * mh), Qn,
                                (((1,), (1,)), ((), ()))) * _SCALE
        s = jnp.where(Ht > 0, s, neg_inf)
        rmax = jnp.max(s, axis=1, keepdims=True)
        rmax = jnp.where(jnp.isfinite(rmax), rmax, 0.0)
        ex = jnp.exp(s - rmax)
        rsum = jnp.sum(ex, axis=1, keepdims=True)
        a = ex / jnp.where(rsum > 0.0, rsum, 1.0)
        he_upd = he_upd + (a @ Vn) * mh
    new_he = he_upd @ wo1_ref[...]

    Q2 = q @ wq2_ref[...]
    K2 = new_he @ wk2_ref[...]
    V2 = new_he @ wv2_ref[...]
    node_upd = jnp.zeros_like(Q2)
    for h in range(_HEADS):
        mh = (hsel == h).astype(jnp.float32)
        s = jax.lax.dot_general(Q2, (K2 * mh),
                                (((1,), (1,)), ((), ()))) * _SCALE
        s = jnp.where(H > 0, s, neg_inf)
        rmax = jnp.max(s, axis=1, keepdims=True)
        rmax = jnp.where(jnp.isfinite(rmax), rmax, 0.0)
        ex = jnp.exp(s - rmax)
        rsum = jnp.sum(ex, axis=1, keepdims=True)
        a = ex / jnp.where(rsum > 0.0, rsum, 1.0)
        node_upd = node_upd + a @ (V2 * mh)
    node_msg = node_upd @ wo2_ref[...]

    hh = node_msg @ wo_ref[...] + q @ wres_ref[...]
    mu = jnp.mean(hh, axis=0, keepdims=True)
    var = jnp.mean((hh - mu) * (hh - mu), axis=0, keepdims=True)
    hh = (hh - mu) / jnp.sqrt(var + 1e-5) * g1_ref[...] + b1_ref[...]
    h_in = hh
    t = hh @ wffn1_ref[...] + bffn1_ref[...]
    t = 0.5 * t * (1.0 + jax.lax.erf(t * (2.0 ** -0.5)))
    t = t @ wffn2_ref[...] + bffn2_ref[...]
    hh = t + h_in
    mu = jnp.mean(hh, axis=0, keepdims=True)
    var = jnp.mean((hh - mu) * (hh - mu), axis=0, keepdims=True)
    out_ref[...] = (hh - mu) / jnp.sqrt(var + 1e-5) * g2_ref[...] + b2_ref[...]


def kernel(graph, q, k, v, edge_feat, H, W_e2i, W_n2h_q, W_n2h_k, W_n2h_v,
           W_n2h_o, W_h2n_q, W_h2n_k, W_h2n_v, W_h2n_o, W_o, W_ffn1, b_ffn1,
           W_ffn2, b_ffn2, W_res, bn1_g, bn1_b, bn2_g, bn2_b):
    num_nodes = q.shape[0]
    oc = W_n2h_q.shape[1]
    return pl.pallas_call(
        _hgt_kernel,
        out_shape=jax.ShapeDtypeStruct((num_nodes, oc), jnp.float32),
    )(q, k, H, H.T, edge_feat,
      W_e2i, W_n2h_q, W_n2h_k, W_n2h_v, W_n2h_o,
      W_h2n_q, W_h2n_k, W_h2n_v, W_h2n_o,
      W_o, W_ffn1, b_ffn1.reshape(1, -1), W_ffn2, b_ffn2.reshape(1, -1),
      W_res, bn1_g.reshape(1, -1), bn1_b.reshape(1, -1),
      bn2_g.reshape(1, -1), bn2_b.reshape(1, -1))


# speedup vs baseline: 2649.5155x; 1.0981x over previous
import jax
import jax.numpy as jnp
from jax.experimental import pallas as pl

_HEADS = 4
_DH = 16
_SCALE = 1.0 / (_DH ** 0.5)


def _hgt_kernel(q_ref, k_ref, H_ref, Ht_ref, ef_ref,
                we2i_ref, wq1_ref, wk1_ref, wv1_ref, wo1_ref,
                wq2_ref, wk2_ref, wv2_ref, wo2_ref,
                wo_ref, wffn1_ref, bffn1_ref, wffn2_ref, bffn2_ref,
                wres_ref, g1_ref, b1_ref, g2_ref, b2_ref,
                out_ref):
    f32 = jnp.float32
    q = q_ref[...]
    k = k_ref[...]
    oc = wq1_ref.shape[1]
    m = ef_ref.shape[0]
    neg_inf = float("-inf")

    maddT = jnp.where(Ht_ref[...] > 0, 0.0, neg_inf)
    madd = jnp.where(H_ref[...] > 0, 0.0, neg_inf)

    hsel = jax.lax.broadcasted_iota(jnp.int32, (_HEADS, 1, oc), 2) // _DH
    hid = jax.lax.broadcasted_iota(jnp.int32, (_HEADS, 1, oc), 0)
    mh3 = (hsel == hid).astype(f32)

    ef = ef_ref[...] @ we2i_ref[...]
    khw = (ef @ wk1_ref[...]) * _SCALE

    kqv_w = jnp.concatenate([wq1_ref[...], wv1_ref[...]], axis=1)
    KQV = k @ kqv_w
    khw4 = khw[None, :, :] * mh3
    khw4p = jnp.concatenate([khw4, jnp.zeros_like(khw4)], axis=2)
    s = jax.lax.dot_general(khw4p.reshape(_HEADS * m, 2 * oc), KQV,
                            (((1,), (1,)), ((), ())))
    s3 = s.reshape(_HEADS, m, -1) + maddT[None, :, :]
    rmax = jnp.maximum(jnp.max(s3, axis=2, keepdims=True), -1e30)
    ex3 = jnp.exp(s3 - rmax)
    rsum = jnp.sum(ex3, axis=2, keepdims=True)
    rinv = jnp.where(rsum > 0.0, 1.0 / rsum, 0.0)
    P = jax.lax.dot_general(ex3.reshape(_HEADS * m, -1), KQV,
                            (((1,), (0,)), ((), ())))
    Vpart = P.reshape(_HEADS, m, 2 * oc)[:, :, oc:]
    he_upd = jnp.sum(Vpart * rinv * mh3, axis=0)
    new_he = he_upd @ wo1_ref[...]

    q2_w = jnp.concatenate([wq2_ref[...], wres_ref[...]], axis=1)
    Q2res = q @ q2_w
    kv2_w = jnp.concatenate([wk2_ref[...] * _SCALE, wv2_ref[...]], axis=1)
    K2V2 = new_he @ kv2_w
    K2 = K2V2[:, :oc]
    V2 = K2V2[:, oc:]
    node_upd = None
    for h in range(_HEADS):
        mh = mh3[h]
        k2p = jnp.concatenate([K2 * mh, jnp.zeros_like(K2)], axis=1)
        sh = jax.lax.dot_general(Q2res, k2p,
                                 (((1,), (1,)), ((), ()))) + madd
        rmax2 = jnp.maximum(jnp.max(sh, axis=1, keepdims=True), -1e30)
        exh = jnp.exp(sh - rmax2)
        rsum2 = jnp.sum(exh, axis=1, keepdims=True)
        rinv2 = jnp.where(rsum2 > 0.0, 1.0 / rsum2, 0.0)
        u = (exh @ (V2 * mh)) * rinv2
        node_upd = u if node_upd is None else node_upd + u
    node_msg = node_upd @ wo2_ref[...]

    hh = node_msg @ wo_ref[...] + Q2res[:, oc:]
    n_inv = 1.0 / hh.shape[0]
    mu = jnp.sum(hh, axis=0, keepdims=True) * n_inv
    msq = jnp.sum(hh * hh, axis=0, keepdims=True) * n_inv
    sc1 = g1_ref[...] / jnp.sqrt(msq - mu * mu + 1e-5)
    hh = hh * sc1 + (b1_ref[...] - mu * sc1)
    h_in = hh
    t = hh @ wffn1_ref[...] + bffn1_ref[...]
    t = 0.5 * t * (1.0 + jax.lax.erf(t * (2.0 ** -0.5)))
    hh = (t @ wffn2_ref[...] + bffn2_ref[...]) + h_in
    mu = jnp.sum(hh, axis=0, keepdims=True) * n_inv
    msq = jnp.sum(hh * hh, axis=0, keepdims=True) * n_inv
    sc2 = g2_ref[...] / jnp.sqrt(msq - mu * mu + 1e-5)
    out_ref[...] = hh * sc2 + (b2_ref[...] - mu * sc2)


def kernel(graph, q, k, v, edge_feat, H, W_e2i, W_n2h_q, W_n2h_k, W_n2h_v,
           W_n2h_o, W_h2n_q, W_h2n_k, W_h2n_v, W_h2n_o, W_o, W_ffn1, b_ffn1,
           W_ffn2, b_ffn2, W_res, bn1_g, bn1_b, bn2_g, bn2_b):
    num_nodes = q.shape[0]
    oc = W_n2h_q.shape[1]
    return pl.pallas_call(
        _hgt_kernel,
        out_shape=jax.ShapeDtypeStruct((num_nodes, oc), jnp.float32),
    )(q, k, H, H.T, edge_feat,
      W_e2i, W_n2h_q, W_n2h_k, W_n2h_v, W_n2h_o,
      W_h2n_q, W_h2n_k, W_h2n_v, W_h2n_o,
      W_o, W_ffn1, b_ffn1.reshape(1, -1), W_ffn2, b_ffn2.reshape(1, -1),
      W_res, bn1_g.reshape(1, -1), bn1_b.reshape(1, -1),
      bn2_g.reshape(1, -1), bn2_b.reshape(1, -1))


# speedup vs baseline: 3271.1496x; 1.2346x over previous
import jax
import jax.numpy as jnp
from jax.experimental import pallas as pl

_HEADS = 4
_DH = 16
_SCALE = 1.0 / (_DH ** 0.5)


def _hgt_kernel(q_ref, k_ref, Ht_ref, ef_ref,
                we2i_ref, wq1_ref, wk1_ref, wv1_ref, wo1_ref,
                wq2_ref, wk2_ref, wv2_ref, wo2_ref,
                wo_ref, wffn1_ref, bffn1_ref, wffn2_ref, bffn2_ref,
                wres_ref, g1_ref, b1_ref, g2_ref, b2_ref,
                out_ref):
    f32 = jnp.float32
    q = q_ref[...]
    k = k_ref[...]
    oc = wq1_ref.shape[1]
    m = ef_ref.shape[0]
    neg_inf = float("-inf")

    maddT = jnp.where(Ht_ref[...] > 0, 0.0, neg_inf)

    hsel = jax.lax.broadcasted_iota(jnp.int32, (_HEADS, 1, oc), 2) // _DH
    hid = jax.lax.broadcasted_iota(jnp.int32, (_HEADS, 1, oc), 0)
    mh3 = (hsel == hid).astype(f32)
    hselr = jax.lax.broadcasted_iota(jnp.int32, (_HEADS, oc, 1), 1) // _DH
    hidr = jax.lax.broadcasted_iota(jnp.int32, (_HEADS, oc, 1), 0)
    mhr = (hselr == hidr).astype(f32)

    ef = ef_ref[...] @ we2i_ref[...]
    khw = (ef @ wk1_ref[...]) * _SCALE

    kqv_w = jnp.concatenate([wq1_ref[...], wv1_ref[...]], axis=1)
    KQV = k @ kqv_w
    khw4 = khw[None, :, :] * mh3
    khw4p = jnp.concatenate([khw4, jnp.zeros_like(khw4)], axis=2)
    s = jax.lax.dot_general(khw4p.reshape(_HEADS * m, 2 * oc), KQV,
                            (((1,), (1,)), ((), ())))
    s3 = s.reshape(_HEADS, m, -1) + maddT[None, :, :]
    rmax = jnp.maximum(jnp.max(s3, axis=2, keepdims=True), -1e30)
    ex3 = jnp.exp(s3 - rmax)
    rsum = jnp.sum(ex3, axis=2, keepdims=True)
    rinv = jnp.where(rsum > 0.0, 1.0 / rsum, 0.0)
    P = jax.lax.dot_general(ex3.reshape(_HEADS * m, -1), KQV,
                            (((1,), (0,)), ((), ())))
    Vpart = P.reshape(_HEADS, m, 2 * oc)[:, :, oc:]
    he_upd = jnp.sum(Vpart * rinv * mh3, axis=0)
    new_he = he_upd @ wo1_ref[...]

    Q2 = q @ wq2_ref[...]
    K2 = new_he @ (wk2_ref[...] * _SCALE)
    V2T = jax.lax.dot_general(wv2_ref[...], new_he,
                              (((0,), (1,)), ((), ())))
    K2stack = (K2[None, :, :] * mh3).reshape(_HEADS * m, oc)
    s2 = jax.lax.dot_general(K2stack, Q2,
                             (((1,), (1,)), ((), ())))
    s23 = s2.reshape(_HEADS, m, -1) + maddT[None, :, :]
    cmax = jnp.maximum(jnp.max(s23, axis=1, keepdims=True), -1e30)
    ex2 = jnp.exp(s23 - cmax)
    csum = jnp.sum(ex2, axis=1, keepdims=True)
    rinv2 = jnp.where(csum > 0.0, 1.0 / csum, 0.0)
    a2 = (ex2 * rinv2).reshape(_HEADS * m, -1)
    V2Tstack = jnp.concatenate([V2T * mhr[h] for h in range(_HEADS)],
                               axis=1)
    node_updT = jax.lax.dot_general(V2Tstack, a2,
                                    (((1,), (0,)), ((), ())))

    node_msgT = jax.lax.dot_general(wo2_ref[...], node_updT,
                                    (((0,), (0,)), ((), ())))
    resT = jax.lax.dot_general(wres_ref[...], q,
                               (((0,), (1,)), ((), ())))
    hhT = jax.lax.dot_general(wo_ref[...], node_msgT,
                              (((0,), (0,)), ((), ()))) + resT
    n_inv = 1.0 / hhT.shape[1]
    mu = jnp.sum(hhT, axis=1, keepdims=True) * n_inv
    msq = jnp.sum(hhT * hhT, axis=1, keepdims=True) * n_inv
    sc1 = g1_ref[...] / jnp.sqrt(msq - mu * mu + 1e-5)
    hhT = hhT * sc1 + (b1_ref[...] - mu * sc1)
    hT_in = hhT
    tT = jax.lax.dot_general(wffn1_ref[...], hhT,
                             (((0,), (0,)), ((), ()))) + bffn1_ref[...]
    tT = 0.5 * tT * (1.0 + jax.lax.erf(tT * (2.0 ** -0.5)))
    hhT = jax.lax.dot_general(wffn2_ref[...], tT,
                              (((0,), (0,)), ((), ()))) + bffn2_ref[...]
    hhT = hhT + hT_in
    mu = jnp.sum(hhT, axis=1, keepdims=True) * n_inv
    msq = jnp.sum(hhT * hhT, axis=1, keepdims=True) * n_inv
    sc2 = g2_ref[...] / jnp.sqrt(msq - mu * mu + 1e-5)
    outT = hhT * sc2 + (b2_ref[...] - mu * sc2)
    out_ref[...] = outT.T


def kernel(graph, q, k, v, edge_feat, H, W_e2i, W_n2h_q, W_n2h_k, W_n2h_v,
           W_n2h_o, W_h2n_q, W_h2n_k, W_h2n_v, W_h2n_o, W_o, W_ffn1, b_ffn1,
           W_ffn2, b_ffn2, W_res, bn1_g, bn1_b, bn2_g, bn2_b):
    num_nodes = q.shape[0]
    oc = W_n2h_q.shape[1]
    return pl.pallas_call(
        _hgt_kernel,
        out_shape=jax.ShapeDtypeStruct((num_nodes, oc), jnp.float32),
    )(q, k, H.T, edge_feat,
      W_e2i, W_n2h_q, W_n2h_k, W_n2h_v, W_n2h_o,
      W_h2n_q, W_h2n_k, W_h2n_v, W_h2n_o,
      W_o, W_ffn1, b_ffn1.reshape(-1, 1), W_ffn2, b_ffn2.reshape(-1, 1),
      W_res, bn1_g.reshape(-1, 1), bn1_b.reshape(-1, 1),
      bn2_g.reshape(-1, 1), bn2_b.reshape(-1, 1))
